# baseline (speedup 1.0000x reference)
import functools

import jax
import jax.numpy as jnp
from jax import lax
from jax.experimental import pallas as pl
from jax.experimental.pallas import tpu as pltpu
from jax.experimental.pallas import tpu_sc as plsc

N_NODE = 10000
N_EDGE = 2000
N_PAIR = 160000
D = 128

NC = 2
NS = 16
HALF = (N_EDGE // NC) * N_EDGE
CHUNK = N_PAIR // NS
VREGS = CHUNK // 16
IDX_ROWS = (CHUNK + 127) // 128
ZCHUNK = 8000
ZPT = HALF // NS


def _mask_body(p0_hbm, p1_hbm, out_hbm, p0v, p1v, idxv, onesv, zbuf,
               semp, semz, sems):
    cid = lax.axis_index("c")
    sid = lax.axis_index("s")

    hp0 = pltpu.async_copy(p0_hbm.at[pl.ds(sid * CHUNK, CHUNK)], p0v, semp)
    hp1 = pltpu.async_copy(p1_hbm.at[pl.ds(sid * CHUNK, CHUNK)], p1v, semp)

    def _fill_z(i, _):
        zbuf[pl.ds(i * 16, 16)] = jnp.zeros((16,), jnp.float32)
        return 0

    with jax.named_scope("ph_fill"):
        lax.fori_loop(0, ZCHUNK // 16, _fill_z, 0)
        for k in range(8):
            onesv[pl.ds(k * 16, 16)] = jnp.ones((16,), jnp.float32)

    zh = []
    zbase = cid * HALF + sid * ZPT
    nfull, rem = divmod(ZPT, ZCHUNK)
    for k in range(nfull):
        zh.append(pltpu.async_copy(
            zbuf, out_hbm.at[pl.ds(zbase + k * ZCHUNK, ZCHUNK)], semz))
    if rem:
        zh.append(pltpu.async_copy(
            zbuf.at[pl.ds(0, rem)],
            out_hbm.at[pl.ds(zbase + nfull * ZCHUNK, rem)], semz))

    with jax.named_scope("ph_pairwait"):
        hp0.wait()
        hp1.wait()

    def _offsets(i, _):
        p0 = p0v[pl.ds(i * 16, 16)]
        p1 = p1v[pl.ds(i * 16, 16)]
        idxv[i // 8, pl.ds((i % 8) * 16, 16)] = p0 * N_EDGE + p1
        return 0

    with jax.named_scope("ph_offsets"):
        lax.fori_loop(0, VREGS, _offsets, 0)
    if CHUNK % 128:
        pad = p0v[pl.ds(CHUNK - 16, 16)] * N_EDGE + p1v[pl.ds(CHUNK - 16, 16)]
        for k in range(CHUNK % 128 // 16, 8):
            idxv[IDX_ROWS - 1, pl.ds(k * 16, 16)] = pad

    with jax.named_scope("ph_zerowait"):
        for h in zh:
            h.wait()
        plsc.subcore_barrier()

    with jax.named_scope("ph_scatter"):
        sh = [pltpu.async_copy(onesv, out_hbm.at[idxv.at[j]], sems)
              for j in range(IDX_ROWS)]
        for h in sh:
            h.wait()


_build_mask = functools.partial(
    pl.kernel,
    mesh=plsc.VectorSubcoreMesh(core_axis_name="c", subcore_axis_name="s"),
    out_type=jax.ShapeDtypeStruct((N_EDGE * N_EDGE,), jnp.float32),
    scratch_types=[
        pltpu.VMEM((CHUNK,), jnp.int32),
        pltpu.VMEM((CHUNK,), jnp.int32),
        pltpu.VMEM((IDX_ROWS, 128), jnp.int32),
        pltpu.VMEM((128,), jnp.float32),
        pltpu.VMEM((ZCHUNK,), jnp.float32),
        pltpu.SemaphoreType.DMA,
        pltpu.SemaphoreType.DMA,
        pltpu.SemaphoreType.DMA,
    ],
)(_mask_body)


def _proj_body(x_ref, xe_ref, wt_ref, a_ref, xep_ref, xa_ref, xph_ref, ts_ref):
    wt = wt_ref[...]
    xp = jnp.dot(x_ref[...], wt, preferred_element_type=jnp.float32)
    xep_ref[...] = jnp.dot(xe_ref[...], wt, preferred_element_type=jnp.float32)
    xph = xp[:N_EDGE]
    xph_ref[...] = xph
    xa_ref[...] = xph * a_ref[...]
    ts_ref[...] = jnp.sum(xp[N_EDGE:], axis=0, keepdims=True)


_project = pl.pallas_call(
    _proj_body,
    out_shape=[
        jax.ShapeDtypeStruct((N_EDGE, D), jnp.float32),
        jax.ShapeDtypeStruct((N_EDGE, D), jnp.float32),
        jax.ShapeDtypeStruct((N_EDGE, D), jnp.float32),
        jax.ShapeDtypeStruct((1, D), jnp.float32),
    ],
)


def _attn_body(xep_ref, xa_ref, xph_ref, ts_ref, m_ref, xout_ref, xeout_ref):
    xep = xep_ref[...]
    s = lax.dot_general(xep, xa_ref[...], (((1,), (1,)), ((), ())),
                        preferred_element_type=jnp.float32)
    g = 1e-10 + jnp.where(m_ref[...] > 0, jnp.exp(s), 0.0)

    ze = jnp.sum(g, axis=1, keepdims=True) + (N_NODE - N_EDGE) * 1e-10
    pe = g / ze
    xe_out = (jnp.dot(pe, xph_ref[...], preferred_element_type=jnp.float32)
              + (1e-10 / ze) * ts_ref[...])
    xeout_ref[...] = jnp.where(xe_out > 0, xe_out, jnp.exp(xe_out) - 1.0)

    zn = jnp.sum(g, axis=0, keepdims=True)
    qn = g / zn
    x_head = lax.dot_general(qn, xep, (((0,), (0,)), ((), ())),
                             preferred_element_type=jnp.float32)
    xout_ref[:N_EDGE] = jnp.where(x_head > 0, x_head, jnp.exp(x_head) - 1.0)
    x_tail = jnp.sum(xep, axis=0, keepdims=True) * (1.0 / N_EDGE)
    x_tail = jnp.where(x_tail > 0, x_tail, jnp.exp(x_tail) - 1.0)
    xout_ref[N_EDGE:] = jnp.broadcast_to(x_tail, (N_NODE - N_EDGE, D))


_attend = pl.pallas_call(
    _attn_body,
    out_shape=[
        jax.ShapeDtypeStruct((N_NODE, D), jnp.float32),
        jax.ShapeDtypeStruct((N_EDGE, D), jnp.float32),
    ],
)


def kernel(x, xe, pair, a, wt):
    m = _build_mask(pair[0], pair[1]).reshape(N_EDGE, N_EDGE)
    xep, xa, xph, ts = _project(x, xe, wt, a.reshape(1, D))
    x_out, xe_out = _attend(xep, xa, xph, ts, m)
    return x_out, xe_out


# speedup vs baseline: 1.8382x; 1.8382x over previous
import functools

import jax
import jax.numpy as jnp
from jax import lax
from jax.experimental import pallas as pl
from jax.experimental.pallas import tpu as pltpu
from jax.experimental.pallas import tpu_sc as plsc

N_NODE = 10000
N_EDGE = 2000
N_PAIR = 160000
D = 128

NC = 2
NS = 16
ROWS_CORE = N_EDGE // NC
ROWS_P0 = 512
ROWS_P1 = ROWS_CORE - ROWS_P0
P0_WORDS = ROWS_P0 * N_EDGE
P1_WORDS = ROWS_P1 * N_EDGE
DUMP = P0_WORDS
CHUNK = N_PAIR // NS
VREGS = CHUNK // 16
IDX_ROWS = (CHUNK + 127) // 128
ZCHUNK = 8000
CHUNKS0 = [ZCHUNK] * 8
CHUNKS1 = [ZCHUNK] * 7 + [5000]


def _mask_body(p0_hbm, p1_hbm, out_hbm, shared, p0v, p1v, idx0, idx1,
               onesv, zbuf, obuf0, obuf1, semp, semz, sems, semi, semo):
    cid = lax.axis_index("c")
    sid = lax.axis_index("s")

    hp0 = pltpu.async_copy(p0_hbm.at[pl.ds(sid * CHUNK, CHUNK)], p0v, semp)
    hp1 = pltpu.async_copy(p1_hbm.at[pl.ds(sid * CHUNK, CHUNK)], p1v, semp)

    def _fill_z(i, _):
        zbuf[pl.ds(i * 16, 16)] = jnp.zeros((16,), jnp.float32)
        return 0

    with jax.named_scope("ph_fill"):
        lax.fori_loop(0, ZCHUNK // 16, _fill_z, 0)
        for k in range(8):
            onesv[pl.ds(k * 16, 16)] = jnp.ones((16,), jnp.float32)

    def _fire_zero(chunks, wpt):
        hs = []
        base = sid * wpt
        off = 0
        for c in chunks:
            hs.append(pltpu.async_copy(
                zbuf.at[pl.ds(0, c)], shared.at[pl.ds(base + off, c)], semz))
            off += c
        return hs

    zh = _fire_zero(CHUNKS0, sum(CHUNKS0))

    with jax.named_scope("ph_pairwait"):
        hp0.wait()
        hp1.wait()

    def _offsets(i, _):
        p0 = p0v[pl.ds(i * 16, 16)]
        p1 = p1v[pl.ds(i * 16, 16)]
        r = p0 - cid * ROWS_CORE
        off0 = jnp.where((r >= 0) & (r < ROWS_P0), r * N_EDGE + p1, DUMP)
        off1 = jnp.where((r >= ROWS_P0) & (r < ROWS_CORE),
                         (r - ROWS_P0) * N_EDGE + p1, DUMP)
        idx0[i // 8, pl.ds((i % 8) * 16, 16)] = off0
        idx1[i // 8, pl.ds((i % 8) * 16, 16)] = off1
        return 0

    with jax.named_scope("ph_offsets"):
        lax.fori_loop(0, VREGS, _offsets, 0)
        pad = jnp.full((16,), DUMP, jnp.int32)
        for k in range(CHUNK % 128 // 16, 8):
            idx0[IDX_ROWS - 1, pl.ds(k * 16, 16)] = pad
            idx1[IDX_ROWS - 1, pl.ds(k * 16, 16)] = pad

    with jax.named_scope("ph_zerowait"):
        for h in zh:
            h.wait()
    plsc.subcore_barrier()

    def _scatter(idxv):
        with jax.named_scope("ph_scatter"):
            sh = [pltpu.async_copy(onesv, shared.at[idxv.at[j]], sems)
                  for j in range(IDX_ROWS)]
            for h in sh:
                h.wait()
        plsc.subcore_barrier()

    def _copy_out(chunks, wpt, hbm_base):
        obufs = (obuf0, obuf1)
        starts = [sum(chunks[:c]) for c in range(len(chunks))]
        base = sid * wpt
        hin = [None] * len(chunks)
        hout = [None] * len(chunks)

        def _fire_in(c):
            return pltpu.async_copy(
                shared.at[pl.ds(base + starts[c], chunks[c])],
                obufs[c % 2].at[pl.ds(0, chunks[c])], semi)

        with jax.named_scope("ph_copyout"):
            hin[0] = _fire_in(0)
            for c in range(len(chunks)):
                hin[c].wait()
                hout[c] = pltpu.async_copy(
                    obufs[c % 2].at[pl.ds(0, chunks[c])],
                    out_hbm.at[pl.ds(hbm_base + base + starts[c], chunks[c])],
                    semo)
                if c + 1 < len(chunks):
                    if c >= 1:
                        hout[c - 1].wait()
                    hin[c + 1] = _fire_in(c + 1)
        return [h for h in hout[-2:] if h is not None]

    _scatter(idx0)
    pend = _copy_out(CHUNKS0, sum(CHUNKS0), cid * ROWS_CORE * N_EDGE)
    zh = _fire_zero(CHUNKS1, sum(CHUNKS1))
    with jax.named_scope("ph_zerowait"):
        for h in pend + zh:
            h.wait()
    plsc.subcore_barrier()

    _scatter(idx1)
    pend = _copy_out(CHUNKS1, sum(CHUNKS1),
                     cid * ROWS_CORE * N_EDGE + P0_WORDS)
    for h in pend:
        h.wait()


_build_mask = functools.partial(
    pl.kernel,
    mesh=plsc.VectorSubcoreMesh(core_axis_name="c", subcore_axis_name="s"),
    out_type=jax.ShapeDtypeStruct((N_EDGE * N_EDGE,), jnp.float32),
    scratch_types=[
        pltpu.VMEM_SHARED((P0_WORDS + 8,), jnp.float32),
        pltpu.VMEM((CHUNK,), jnp.int32),
        pltpu.VMEM((CHUNK,), jnp.int32),
        pltpu.VMEM((IDX_ROWS, 128), jnp.int32),
        pltpu.VMEM((IDX_ROWS, 128), jnp.int32),
        pltpu.VMEM((128,), jnp.float32),
        pltpu.VMEM((ZCHUNK,), jnp.float32),
        pltpu.VMEM((ZCHUNK,), jnp.float32),
        pltpu.VMEM((ZCHUNK,), jnp.float32),
        pltpu.SemaphoreType.DMA,
        pltpu.SemaphoreType.DMA,
        pltpu.SemaphoreType.DMA,
        pltpu.SemaphoreType.DMA,
        pltpu.SemaphoreType.DMA,
    ],
)(_mask_body)


def _proj_body(x_ref, xe_ref, wt_ref, a_ref, xep_ref, xa_ref, xph_ref, ts_ref):
    wt = wt_ref[...]
    xp = jnp.dot(x_ref[...], wt, preferred_element_type=jnp.float32)
    xep_ref[...] = jnp.dot(xe_ref[...], wt, preferred_element_type=jnp.float32)
    xph = xp[:N_EDGE]
    xph_ref[...] = xph
    xa_ref[...] = xph * a_ref[...]
    ts_ref[...] = jnp.sum(xp[N_EDGE:], axis=0, keepdims=True)


_project = pl.pallas_call(
    _proj_body,
    out_shape=[
        jax.ShapeDtypeStruct((N_EDGE, D), jnp.float32),
        jax.ShapeDtypeStruct((N_EDGE, D), jnp.float32),
        jax.ShapeDtypeStruct((N_EDGE, D), jnp.float32),
        jax.ShapeDtypeStruct((1, D), jnp.float32),
    ],
)


def _attn_body(xep_ref, xa_ref, xph_ref, ts_ref, m_ref, xout_ref, xeout_ref):
    xep = xep_ref[...]
    s = lax.dot_general(xep, xa_ref[...], (((1,), (1,)), ((), ())),
                        preferred_element_type=jnp.float32)
    g = 1e-10 + jnp.where(m_ref[...] > 0, jnp.exp(s), 0.0)

    ze = jnp.sum(g, axis=1, keepdims=True) + (N_NODE - N_EDGE) * 1e-10
    pe = g / ze
    xe_out = (jnp.dot(pe, xph_ref[...], preferred_element_type=jnp.float32)
              + (1e-10 / ze) * ts_ref[...])
    xeout_ref[...] = jnp.where(xe_out > 0, xe_out, jnp.exp(xe_out) - 1.0)

    zn = jnp.sum(g, axis=0, keepdims=True)
    qn = g / zn
    x_head = lax.dot_general(qn, xep, (((0,), (0,)), ((), ())),
                             preferred_element_type=jnp.float32)
    xout_ref[:N_EDGE] = jnp.where(x_head > 0, x_head, jnp.exp(x_head) - 1.0)
    x_tail = jnp.sum(xep, axis=0, keepdims=True) * (1.0 / N_EDGE)
    x_tail = jnp.where(x_tail > 0, x_tail, jnp.exp(x_tail) - 1.0)
    xout_ref[N_EDGE:] = jnp.broadcast_to(x_tail, (N_NODE - N_EDGE, D))


_attend = pl.pallas_call(
    _attn_body,
    out_shape=[
        jax.ShapeDtypeStruct((N_NODE, D), jnp.float32),
        jax.ShapeDtypeStruct((N_EDGE, D), jnp.float32),
    ],
)


def kernel(x, xe, pair, a, wt):
    m = _build_mask(pair[0], pair[1]).reshape(N_EDGE, N_EDGE)
    xep, xa, xph, ts = _project(x, xe, wt, a.reshape(1, D))
    x_out, xe_out = _attend(xep, xa, xph, ts, m)
    return x_out, xe_out


# speedup vs baseline: 4.5807x; 2.4919x over previous
import functools

import jax
import jax.numpy as jnp
from jax import lax
from jax.experimental import pallas as pl
from jax.experimental.pallas import tpu as pltpu
from jax.experimental.pallas import tpu_sc as plsc

N_NODE = 10000
N_EDGE = 2000
N_PAIR = 160000
D = 128

NC = 2
NS = 16
ROWS_CORE = N_EDGE // NC
ROWS_P0 = 512
ROWS_P1 = ROWS_CORE - ROWS_P0
P0_WORDS = ROWS_P0 * N_EDGE
P1_WORDS = ROWS_P1 * N_EDGE
DUMP = P0_WORDS
CHUNK = N_PAIR // NS
VREGS = CHUNK // 16
IDX_ROWS = (CHUNK + 127) // 128
ZCHUNK = 8000
CHUNKS0 = [ZCHUNK] * 8
CHUNKS1 = [ZCHUNK] * 7 + [5000]


def _mask_body(p0_hbm, p1_hbm, out_hbm, shared, p0v, p1v, idx0, idx1,
               onesv, zbuf, obuf0, obuf1, semp, semz, sems, semi, semo):
    cid = lax.axis_index("c")
    sid = lax.axis_index("s")

    hp0 = pltpu.async_copy(p0_hbm.at[pl.ds(sid * CHUNK, CHUNK)], p0v, semp)
    hp1 = pltpu.async_copy(p1_hbm.at[pl.ds(sid * CHUNK, CHUNK)], p1v, semp)

    def _fill_z(i, _):
        zbuf[pl.ds(i * 16, 16)] = jnp.zeros((16,), jnp.float32)
        return 0

    with jax.named_scope("ph_fill"):
        lax.fori_loop(0, ZCHUNK // 16, _fill_z, 0)
        for k in range(8):
            onesv[pl.ds(k * 16, 16)] = jnp.ones((16,), jnp.float32)

    def _fire_zero(chunks, wpt):
        hs = []
        base = sid * wpt
        off = 0
        for c in chunks:
            hs.append(pltpu.async_copy(
                zbuf.at[pl.ds(0, c)], shared.at[pl.ds(base + off, c)], semz))
            off += c
        return hs

    zh = _fire_zero(CHUNKS0, sum(CHUNKS0))

    with jax.named_scope("ph_pairwait"):
        hp0.wait()
        hp1.wait()

    def _offsets(i, _):
        p0 = p0v[pl.ds(i * 16, 16)]
        p1 = p1v[pl.ds(i * 16, 16)]
        r = p0 - cid * ROWS_CORE
        off0 = jnp.where((r >= 0) & (r < ROWS_P0), r * N_EDGE + p1,
                         DUMP + p1)
        off1 = jnp.where((r >= ROWS_P0) & (r < ROWS_CORE),
                         (r - ROWS_P0) * N_EDGE + p1, DUMP + p1)
        idx0[i // 8, pl.ds((i % 8) * 16, 16)] = off0
        idx1[i // 8, pl.ds((i % 8) * 16, 16)] = off1
        return 0

    with jax.named_scope("ph_offsets"):
        lax.fori_loop(0, VREGS, _offsets, 0)
        pad = jnp.full((16,), DUMP, jnp.int32)
        for k in range(CHUNK % 128 // 16, 8):
            idx0[IDX_ROWS - 1, pl.ds(k * 16, 16)] = pad
            idx1[IDX_ROWS - 1, pl.ds(k * 16, 16)] = pad

    with jax.named_scope("ph_zerowait"):
        for h in zh:
            h.wait()
    plsc.subcore_barrier()

    def _scatter(idxv):
        with jax.named_scope("ph_scatter"):
            sh = [pltpu.async_copy(onesv, shared.at[idxv.at[j]], sems)
                  for j in range(IDX_ROWS)]
            for h in sh:
                h.wait()
        plsc.subcore_barrier()

    def _copy_out(chunks, wpt, hbm_base):
        obufs = (obuf0, obuf1)
        starts = [sum(chunks[:c]) for c in range(len(chunks))]
        base = sid * wpt
        hin = [None] * len(chunks)
        hout = [None] * len(chunks)

        def _fire_in(c):
            return pltpu.async_copy(
                shared.at[pl.ds(base + starts[c], chunks[c])],
                obufs[c % 2].at[pl.ds(0, chunks[c])], semi)

        with jax.named_scope("ph_copyout"):
            hin[0] = _fire_in(0)
            for c in range(len(chunks)):
                hin[c].wait()
                hout[c] = pltpu.async_copy(
                    obufs[c % 2].at[pl.ds(0, chunks[c])],
                    out_hbm.at[pl.ds(hbm_base + base + starts[c], chunks[c])],
                    semo)
                if c + 1 < len(chunks):
                    if c >= 1:
                        hout[c - 1].wait()
                    hin[c + 1] = _fire_in(c + 1)
        return [h for h in hout[-2:] if h is not None]

    _scatter(idx0)
    pend = _copy_out(CHUNKS0, sum(CHUNKS0), cid * ROWS_CORE * N_EDGE)
    zh = _fire_zero(CHUNKS1, sum(CHUNKS1))
    with jax.named_scope("ph_zerowait"):
        for h in pend + zh:
            h.wait()
    plsc.subcore_barrier()

    _scatter(idx1)
    pend = _copy_out(CHUNKS1, sum(CHUNKS1),
                     cid * ROWS_CORE * N_EDGE + P0_WORDS)
    for h in pend:
        h.wait()


_build_mask = functools.partial(
    pl.kernel,
    mesh=plsc.VectorSubcoreMesh(core_axis_name="c", subcore_axis_name="s"),
    out_type=jax.ShapeDtypeStruct((N_EDGE * N_EDGE,), jnp.float32),
    scratch_types=[
        pltpu.VMEM_SHARED((P0_WORDS + N_EDGE,), jnp.float32),
        pltpu.VMEM((CHUNK,), jnp.int32),
        pltpu.VMEM((CHUNK,), jnp.int32),
        pltpu.VMEM((IDX_ROWS, 128), jnp.int32),
        pltpu.VMEM((IDX_ROWS, 128), jnp.int32),
        pltpu.VMEM((128,), jnp.float32),
        pltpu.VMEM((ZCHUNK,), jnp.float32),
        pltpu.VMEM((ZCHUNK,), jnp.float32),
        pltpu.VMEM((ZCHUNK,), jnp.float32),
        pltpu.SemaphoreType.DMA,
        pltpu.SemaphoreType.DMA,
        pltpu.SemaphoreType.DMA,
        pltpu.SemaphoreType.DMA,
        pltpu.SemaphoreType.DMA,
    ],
)(_mask_body)


def _proj_body(x_ref, xe_ref, wt_ref, a_ref, xep_ref, xa_ref, xph_ref, ts_ref):
    wt = wt_ref[...]
    xp = jnp.dot(x_ref[...], wt, preferred_element_type=jnp.float32)
    xep_ref[...] = jnp.dot(xe_ref[...], wt, preferred_element_type=jnp.float32)
    xph = xp[:N_EDGE]
    xph_ref[...] = xph
    xa_ref[...] = xph * a_ref[...]
    ts_ref[...] = jnp.sum(xp[N_EDGE:], axis=0, keepdims=True)


_project = pl.pallas_call(
    _proj_body,
    out_shape=[
        jax.ShapeDtypeStruct((N_EDGE, D), jnp.float32),
        jax.ShapeDtypeStruct((N_EDGE, D), jnp.float32),
        jax.ShapeDtypeStruct((N_EDGE, D), jnp.float32),
        jax.ShapeDtypeStruct((1, D), jnp.float32),
    ],
)


def _attn_body(xep_ref, xa_ref, xph_ref, ts_ref, m_ref, xout_ref, xeout_ref):
    xep = xep_ref[...]
    s = lax.dot_general(xep, xa_ref[...], (((1,), (1,)), ((), ())),
                        preferred_element_type=jnp.float32)
    g = 1e-10 + jnp.where(m_ref[...] > 0, jnp.exp(s), 0.0)

    ze = jnp.sum(g, axis=1, keepdims=True) + (N_NODE - N_EDGE) * 1e-10
    pe = g / ze
    xe_out = (jnp.dot(pe, xph_ref[...], preferred_element_type=jnp.float32)
              + (1e-10 / ze) * ts_ref[...])
    xeout_ref[...] = jnp.where(xe_out > 0, xe_out, jnp.exp(xe_out) - 1.0)

    zn = jnp.sum(g, axis=0, keepdims=True)
    qn = g / zn
    x_head = lax.dot_general(qn, xep, (((0,), (0,)), ((), ())),
                             preferred_element_type=jnp.float32)
    xout_ref[:N_EDGE] = jnp.where(x_head > 0, x_head, jnp.exp(x_head) - 1.0)
    x_tail = jnp.sum(xep, axis=0, keepdims=True) * (1.0 / N_EDGE)
    x_tail = jnp.where(x_tail > 0, x_tail, jnp.exp(x_tail) - 1.0)
    xout_ref[N_EDGE:] = jnp.broadcast_to(x_tail, (N_NODE - N_EDGE, D))


_attend = pl.pallas_call(
    _attn_body,
    out_shape=[
        jax.ShapeDtypeStruct((N_NODE, D), jnp.float32),
        jax.ShapeDtypeStruct((N_EDGE, D), jnp.float32),
    ],
)


def kernel(x, xe, pair, a, wt):
    m = _build_mask(pair[0], pair[1]).reshape(N_EDGE, N_EDGE)
    xep, xa, xph, ts = _project(x, xe, wt, a.reshape(1, D))
    x_out, xe_out = _attend(xep, xa, xph, ts, m)
    return x_out, xe_out
